# baseline (speedup 1.0000x reference)
import functools

import jax
import jax.numpy as jnp
from jax import lax
from jax.experimental import pallas as pl
from jax.experimental.pallas import tpu as pltpu
from jax.experimental.pallas import tpu_sc as plsc

_NUM_CLASSES = 1000000
_FEAT_DIM = 64
_BATCH = 16384

_NC = 2
_NS = 16
_NW = _NC * _NS
_LANES = 16

_B_PER_W = _BATCH // _NW
_IDX_CHUNK = 128
_NCHUNK = _B_PER_W // _IDX_CHUNK
_JSTEPS = _FEAT_DIM // _LANES


def _sc_center_loss_body(labels_hbm, feats_hbm, centers_hbm, out_hbm,
                         idx_v, rows_v, feats_v, acc_v, sem):
    wid = lax.axis_index("s") * _NC + lax.axis_index("c")
    base = wid * _B_PER_W

    pltpu.sync_copy(labels_hbm.at[wid], idx_v)
    pltpu.sync_copy(feats_hbm.at[pl.ds(base, _B_PER_W)], feats_v)

    copies = []
    for c in range(_NCHUNK):
        copies.append(
            pltpu.async_copy(
                centers_hbm.at[idx_v.at[c]],
                rows_v.at[pl.ds(c * _IDX_CHUNK, _IDX_CHUNK)],
                sem,
            )
        )
    for cp in copies:
        cp.wait()

    def row_body(i, accs):
        new = []
        for j in range(_JSTEPS):
            f = feats_v[i, pl.ds(j * _LANES, _LANES)]
            cc = rows_v[i, pl.ds(j * _LANES, _LANES)]
            d = f - cc
            new.append(accs[j] + d * d)
        return tuple(new)

    zeros = tuple(jnp.zeros((_LANES,), jnp.float32) for _ in range(_JSTEPS))
    accs = lax.fori_loop(0, _B_PER_W, row_body, zeros)
    total = accs[0] + accs[1] + accs[2] + accs[3]
    acc_v[...] = total
    pltpu.sync_copy(acc_v, out_hbm.at[wid])


_sc_center_loss = functools.partial(
    pl.kernel,
    out_type=jax.ShapeDtypeStruct((_NW, _LANES), jnp.float32),
    mesh=plsc.VectorSubcoreMesh(core_axis_name="c", subcore_axis_name="s"),
    compiler_params=pltpu.CompilerParams(use_tc_tiling_on_sc=False),
    scratch_types=[
        pltpu.VMEM((_NCHUNK, _IDX_CHUNK), jnp.int32),
        pltpu.VMEM((_B_PER_W, _FEAT_DIM), jnp.float32),
        pltpu.VMEM((_B_PER_W, _FEAT_DIM), jnp.float32),
        pltpu.VMEM((_LANES,), jnp.float32),
        pltpu.SemaphoreType.DMA,
    ],
)(_sc_center_loss_body)


def kernel(features, labels, centers):
    labels3 = labels.astype(jnp.int32).reshape(_NW, _NCHUNK, _IDX_CHUNK)
    partials = _sc_center_loss(labels3, features, centers)
    return jnp.sum(partials) / _BATCH


# speedup vs baseline: 1.6818x; 1.6818x over previous
import functools

import jax
import jax.numpy as jnp
from jax import lax
from jax.experimental import pallas as pl
from jax.experimental.pallas import tpu as pltpu
from jax.experimental.pallas import tpu_sc as plsc

_NUM_CLASSES = 1000000
_FEAT_DIM = 64
_BATCH = 16384

_NC = 2
_NS = 16
_NW = _NC * _NS
_LANES = 16

_B_PER_W = _BATCH // _NW
_NGROUPS = _B_PER_W // _LANES
_JSTEPS = _FEAT_DIM // _LANES


_CHUNK = 128
_NCHUNK = _B_PER_W // _CHUNK
_CGROUPS = _CHUNK // _LANES


def _sc_center_loss_body(labels_hbm, feats_hbm, centers_hbm, out_hbm,
                         lab_v, feats_v, rows0, rows1, acc_v, sem0, sem1):
    wid = lax.axis_index("s") * _NC + lax.axis_index("c")
    base = wid * _B_PER_W

    pltpu.sync_copy(labels_hbm.at[wid], lab_v)
    pltpu.sync_copy(feats_hbm.at[pl.ds(base, _B_PER_W)], feats_v)

    bufs = (rows0, rows1)
    sems = (sem0, sem1)

    def fire_chunk(c):
        buf = bufs[c % 2]
        sem = sems[c % 2]

        def fire_group(g, carry):
            lv = lab_v[pl.ds(c * _CHUNK + g * _LANES, _LANES)]
            for l in range(_LANES):
                pltpu.async_copy(
                    centers_hbm.at[lv[l]],
                    buf.at[g * _LANES + l],
                    sem,
                )
            return carry

        lax.fori_loop(0, _CGROUPS, fire_group, 0)

    def drain_chunk(c):
        pltpu.make_async_copy(
            centers_hbm.at[pl.ds(0, _CHUNK)], bufs[c % 2], sems[c % 2]
        ).wait()

    fire_chunk(0)
    accs = tuple(jnp.zeros((_LANES,), jnp.float32) for _ in range(_JSTEPS))
    for c in range(_NCHUNK):
        drain_chunk(c)
        if c + 1 < _NCHUNK:
            fire_chunk(c + 1)
        buf = bufs[c % 2]

        def row_body(i, accs, buf=buf, c=c):
            new = []
            for j in range(_JSTEPS):
                f = feats_v[c * _CHUNK + i, pl.ds(j * _LANES, _LANES)]
                cc = buf[i, pl.ds(j * _LANES, _LANES)]
                d = f - cc
                new.append(accs[j] + d * d)
            return tuple(new)

        accs = lax.fori_loop(0, _CHUNK, row_body, accs)

    total = accs[0] + accs[1] + accs[2] + accs[3]
    acc_v[...] = total
    pltpu.sync_copy(acc_v, out_hbm.at[wid])


_sc_center_loss = functools.partial(
    pl.kernel,
    out_type=jax.ShapeDtypeStruct((_NW, _LANES), jnp.float32),
    mesh=plsc.VectorSubcoreMesh(core_axis_name="c", subcore_axis_name="s"),
    scratch_types=[
        pltpu.VMEM((_B_PER_W,), jnp.int32),
        pltpu.VMEM((_B_PER_W, _FEAT_DIM), jnp.float32),
        pltpu.VMEM((_CHUNK, _FEAT_DIM), jnp.float32),
        pltpu.VMEM((_CHUNK, _FEAT_DIM), jnp.float32),
        pltpu.VMEM((_LANES,), jnp.float32),
        pltpu.SemaphoreType.DMA,
        pltpu.SemaphoreType.DMA,
    ],
)(_sc_center_loss_body)


def kernel(features, labels, centers):
    labels2 = labels.astype(jnp.int32).reshape(_NW, _B_PER_W)
    partials = _sc_center_loss(labels2, features, centers)
    return jnp.sum(partials) / _BATCH


# speedup vs baseline: 1.6921x; 1.0062x over previous
import functools

import jax
import jax.numpy as jnp
from jax import lax
from jax.experimental import pallas as pl
from jax.experimental.pallas import tpu as pltpu
from jax.experimental.pallas import tpu_sc as plsc

_NUM_CLASSES = 1000000
_FEAT_DIM = 64
_BATCH = 16384

_NC = 2
_NS = 16
_NW = _NC * _NS
_LANES = 16

_B_PER_W = _BATCH // _NW
_JSTEPS = _FEAT_DIM // _LANES

_CHUNK = 128
_NCHUNK = _B_PER_W // _CHUNK
_CGROUPS = _CHUNK // _LANES


def _sc_center_loss_body(labels_hbm, feats_hbm, centers_hbm, out_hbm,
                         lab_v, feat0, feat1, tile0, tile1, acc_v,
                         semf0, semf1, semt0, semt1):
    wid = lax.axis_index("s") * _NC + lax.axis_index("c")
    base = wid * _B_PER_W

    pltpu.sync_copy(labels_hbm.at[wid], lab_v)

    fbufs = (feat0, feat1)
    tbufs = (tile0, tile1)
    fsems = (semf0, semf1)
    tsems = (semt0, semt1)

    def fire(c):
        hf = pltpu.async_copy(
            feats_hbm.at[pl.ds(base + c * _CHUNK, _CHUNK)],
            fbufs[c % 2], fsems[c % 2],
        )
        buf = tbufs[c % 2]
        sem = tsems[c % 2]

        def fire_group(g, carry):
            lv = lab_v[pl.ds(c * _CHUNK + g * _LANES, _LANES)]
            for l in range(_LANES):
                pltpu.async_copy(
                    centers_hbm.at[lv[l]],
                    buf.at[g * _LANES + l],
                    sem,
                )
            return carry

        lax.fori_loop(0, _CGROUPS, fire_group, 0)
        return hf

    def drain_rows(c):
        pltpu.make_async_copy(
            centers_hbm.at[pl.ds(0, _CHUNK)], tbufs[c % 2], tsems[c % 2]
        ).wait()

    handles = [None] * _NCHUNK
    handles[0] = fire(0)
    accs = tuple(jnp.zeros((_LANES,), jnp.float32) for _ in range(_JSTEPS))
    for c in range(_NCHUNK):
        if c + 1 < _NCHUNK:
            handles[c + 1] = fire(c + 1)
        handles[c].wait()
        drain_rows(c)
        fbuf = fbufs[c % 2]
        tbuf = tbufs[c % 2]

        def row_body(i, accs, fbuf=fbuf, tbuf=tbuf):
            new = []
            for j in range(_JSTEPS):
                f = fbuf[i, pl.ds(j * _LANES, _LANES)]
                cc = tbuf[i, pl.ds(j * _LANES, _LANES)]
                d = f - cc
                new.append(accs[j] + d * d)
            return tuple(new)

        accs = lax.fori_loop(0, _CHUNK, row_body, accs)

    total = accs[0] + accs[1] + accs[2] + accs[3]
    acc_v[...] = total
    pltpu.sync_copy(acc_v, out_hbm.at[wid])


_sc_center_loss = functools.partial(
    pl.kernel,
    out_type=jax.ShapeDtypeStruct((_NW, _LANES), jnp.float32),
    mesh=plsc.VectorSubcoreMesh(core_axis_name="c", subcore_axis_name="s"),
    scratch_types=[
        pltpu.VMEM((_B_PER_W,), jnp.int32),
        pltpu.VMEM((_CHUNK, _FEAT_DIM), jnp.float32),
        pltpu.VMEM((_CHUNK, _FEAT_DIM), jnp.float32),
        pltpu.VMEM((_CHUNK, _FEAT_DIM), jnp.float32),
        pltpu.VMEM((_CHUNK, _FEAT_DIM), jnp.float32),
        pltpu.VMEM((_LANES,), jnp.float32),
        pltpu.SemaphoreType.DMA,
        pltpu.SemaphoreType.DMA,
        pltpu.SemaphoreType.DMA,
        pltpu.SemaphoreType.DMA,
    ],
)(_sc_center_loss_body)


def kernel(features, labels, centers):
    labels2 = labels.astype(jnp.int32).reshape(_NW, _B_PER_W)
    partials = _sc_center_loss(labels2, features, centers)
    return jnp.sum(partials) / _BATCH
